# Initial kernel scaffold; baseline (speedup 1.0000x reference)
import jax
import jax.numpy as jnp
from jax.experimental import pallas as pl


def kernel(x, edge_index, edge_attr, We, be, W1, b1, g1, bt1, W2, b2, Wc1, bc1, Wc2, bc2):
    raise NotImplementedError("write your pallas kernel here")


# speedup vs baseline: 1.8255x; 1.8255x over previous
import jax
import jax.numpy as jnp
from jax import lax
from jax.experimental import pallas as pl
from jax.experimental.pallas import tpu as pltpu
from jax.experimental.pallas import tpu_sc as plsc

N = 10000
E = 160000
D = 256
ED = 16
HALF = 128
OUT = 128
L = 3
BN_EPS = 1e-5

NC = 2
NS = 16
EPT = E // NS
CH = 80
NCH = EPT // CH
RPT = N // NS
RCH = 125
NRC = RPT // RCH


def _edge_agg_body(x2, e2, src, dst, out, idx_s, idx_d, xrows, erows, ibuf,
                   agg, sem):
    c = lax.axis_index("c")
    s = lax.axis_index("s")
    cN = c * N

    for j in range(NRC):
        base = s * RPT + j * RCH
        pltpu.sync_copy(x2.at[pl.ds(cN + base, RCH)], ibuf)
        pltpu.sync_copy(ibuf, agg.at[pl.ds(base, RCH)])
    plsc.subcore_barrier()

    def chunk(jc, carry):
        ebase = s * EPT + jc * CH
        pltpu.sync_copy(src.at[pl.ds(ebase, CH)], idx_s)
        pltpu.sync_copy(dst.at[pl.ds(ebase, CH)], idx_d)
        for v in range(CH // 16):
            sl = pl.ds(v * 16, 16)
            idx_s[sl] = idx_s[sl] + cN
        pltpu.async_copy(x2.at[idx_s], xrows, sem).wait()
        pltpu.sync_copy(e2.at[pl.ds(c * E + ebase, CH)], erows)

        def row(r, rc):
            for v in range(HALF // 16):
                sl = pl.ds(v * 16, 16)
                erows[r, sl] = jnp.maximum(erows[r, sl] + xrows[r, sl], 0.0)
            return rc

        lax.fori_loop(0, CH, row, 0)
        pltpu.sync_copy(erows, agg.at[idx_d], add=True)
        return carry

    lax.fori_loop(0, NCH, chunk, 0)
    plsc.subcore_barrier()

    for j in range(NRC):
        base = s * RPT + j * RCH
        pltpu.sync_copy(agg.at[pl.ds(base, RCH)], ibuf)
        pltpu.sync_copy(ibuf, out.at[pl.ds(cN + base, RCH)])


_EDGE_AGG_CACHE = []


def _edge_agg(h2, e2, src, dst):
    if not _EDGE_AGG_CACHE:
        _EDGE_AGG_CACHE.append(pl.kernel(
            _edge_agg_body,
            out_type=jax.ShapeDtypeStruct((NC * N, HALF), jnp.float32),
            mesh=plsc.VectorSubcoreMesh(core_axis_name="c",
                                        subcore_axis_name="s",
                                        num_cores=NC, num_subcores=NS),
            scratch_types=[
                pltpu.VMEM((CH,), jnp.int32),
                pltpu.VMEM((CH,), jnp.int32),
                pltpu.VMEM((CH, HALF), jnp.float32),
                pltpu.VMEM((CH, HALF), jnp.float32),
                pltpu.VMEM((RCH, HALF), jnp.float32),
                pltpu.VMEM_SHARED((N, HALF), jnp.float32),
                pltpu.SemaphoreType.DMA,
            ],
            compiler_params=pltpu.CompilerParams(use_tc_tiling_on_sc=False),
        ))
    return _EDGE_AGG_CACHE[0](h2, e2, src, dst)


BE = 2000
RB = 1000


def _eproj_body(ea_ref, w_ref, b_ref, out_ref):
    out_ref[0] = (
        jnp.dot(ea_ref[...], w_ref[0], preferred_element_type=jnp.float32)
        + b_ref[0]
    )


def _eproj(ea, w_split, b_split):
    return pl.pallas_call(
        _eproj_body,
        grid=(NC, E // BE),
        in_specs=[
            pl.BlockSpec((BE, ED), lambda c, i: (i, 0)),
            pl.BlockSpec((1, ED, HALF), lambda c, i: (c, 0, 0)),
            pl.BlockSpec((1, 1, HALF), lambda c, i: (c, 0, 0)),
        ],
        out_specs=pl.BlockSpec((1, BE, HALF), lambda c, i: (c, i, 0)),
        out_shape=jax.ShapeDtypeStruct((NC, E, HALF), jnp.float32),
    )(ea, w_split, b_split)


def _mlp_body(xa_ref, w1_ref, b1_ref, sc_ref, bt_ref, w2_ref, b2_ref,
              out_ref):
    h = jnp.concatenate([xa_ref[0], xa_ref[1]], axis=1)
    t = jnp.dot(h, w1_ref[...], preferred_element_type=jnp.float32) + b1_ref[...]
    t = t * sc_ref[...] + bt_ref[...]
    t = jnp.maximum(t, 0.0)
    t = jnp.dot(t, w2_ref[...], preferred_element_type=jnp.float32) + b2_ref[...]
    t = jnp.maximum(t, 0.0)
    out_ref[0] = t[:, :HALF]
    out_ref[1] = t[:, HALF:]


def _mlp(xa, w1, b1r, scr, btr, w2, b2r):
    return pl.pallas_call(
        _mlp_body,
        grid=(N // RB,),
        in_specs=[
            pl.BlockSpec((NC, RB, HALF), lambda i: (0, i, 0)),
            pl.BlockSpec((D, D), lambda i: (0, 0)),
            pl.BlockSpec((1, D), lambda i: (0, 0)),
            pl.BlockSpec((1, D), lambda i: (0, 0)),
            pl.BlockSpec((1, D), lambda i: (0, 0)),
            pl.BlockSpec((D, D), lambda i: (0, 0)),
            pl.BlockSpec((1, D), lambda i: (0, 0)),
        ],
        out_specs=pl.BlockSpec((NC, RB, HALF), lambda i: (0, i, 0)),
        out_shape=jax.ShapeDtypeStruct((NC, N, HALF), jnp.float32),
    )(xa, w1, b1r, scr, btr, w2, b2r)


def _clf_body(xa_ref, w1_ref, b1_ref, w2_ref, b2_ref, out_ref):
    h = jnp.concatenate([xa_ref[0], xa_ref[1]], axis=1)
    t = jnp.dot(h, w1_ref[...], preferred_element_type=jnp.float32) + b1_ref[...]
    t = jnp.maximum(t, 0.0)
    out_ref[...] = (
        jnp.dot(t, w2_ref[...], preferred_element_type=jnp.float32) + b2_ref[...]
    )


def _clf(xa, wc1, bc1r, wc2, bc2r):
    return pl.pallas_call(
        _clf_body,
        grid=(N // RB,),
        in_specs=[
            pl.BlockSpec((NC, RB, HALF), lambda i: (0, i, 0)),
            pl.BlockSpec((D, D), lambda i: (0, 0)),
            pl.BlockSpec((1, D), lambda i: (0, 0)),
            pl.BlockSpec((D, OUT), lambda i: (0, 0)),
            pl.BlockSpec((1, OUT), lambda i: (0, 0)),
        ],
        out_specs=pl.BlockSpec((RB, OUT), lambda i: (i, 0)),
        out_shape=jax.ShapeDtypeStruct((N, OUT), jnp.float32),
    )(xa, wc1, bc1r, wc2, bc2r)


def kernel(x, edge_index, edge_attr, We, be, W1, b1, g1, bt1, W2, b2,
           Wc1, bc1, Wc2, bc2):
    src = edge_index[0].astype(jnp.int32)
    dst = edge_index[1].astype(jnp.int32)
    scale = g1 / jnp.sqrt(1.0 + BN_EPS)

    h2 = x.reshape(N, NC, HALF).transpose(1, 0, 2).reshape(NC * N, HALF)
    for l in range(L):
        w_split = We[l].reshape(ED, NC, HALF).transpose(1, 0, 2)
        b_split = be[l].reshape(NC, 1, HALF)
        e2 = _eproj(edge_attr, w_split, b_split).reshape(NC * E, HALF)
        xa = _edge_agg(h2, e2, src, dst)
        h2 = _mlp(xa.reshape(NC, N, HALF), W1[l], b1[l][None], scale[l][None],
                  bt1[l][None], W2[l], b2[l][None]).reshape(NC * N, HALF)
    return _clf(h2.reshape(NC, N, HALF), Wc1, bc1[None], Wc2, bc2[None])
